# baseline (speedup 1.0000x reference)
import functools

import jax
import jax.numpy as jnp
from jax import lax
from jax.experimental import pallas as pl
from jax.experimental.pallas import tpu as pltpu
from jax.experimental.pallas import tpu_sc as plsc

_NC = 2
_NS = 16
_NW = _NC * _NS
_L = 16

_B = 1000
_BP = 1024
_BW = _BP // _NW
_NNB = 8
_NBW = _BW * _NNB
_NBH = _NBW // 2
_KH = _NNB // 2
_DIM = 512
_EPS = 0.01
_NRELP = 64
_NCTX = 16
_EPR = 16


def _sc_gather(u_pad, v_pad, vg_pad, vm_pad, adj_e8, adj_r8, usr, ent):
    mesh = plsc.VectorSubcoreMesh(core_axis_name="c", subcore_axis_name="s")

    @functools.partial(
        pl.kernel,
        mesh=mesh,
        compiler_params=pltpu.CompilerParams(needs_layout_passes=False),
        out_type=(
            jax.ShapeDtypeStruct((_BP, _DIM), jnp.float32),
            jax.ShapeDtypeStruct((_BP, _DIM), jnp.float32),
            jax.ShapeDtypeStruct((_BP * _NNB,), jnp.int32),
            jax.ShapeDtypeStruct((_NNB, _BP, _DIM), jnp.float32),
        ),
        scratch_types=[
            pltpu.VMEM((_BW,), jnp.int32),
            pltpu.VMEM((_BW,), jnp.int32),
            pltpu.VMEM((_BW,), jnp.int32),
            pltpu.VMEM((_BW,), jnp.int32),
            pltpu.VMEM((_BW, 128), jnp.int32),
            pltpu.VMEM((_BW, 128), jnp.int32),
            pltpu.VMEM((_BW, _DIM), jnp.float32),
            pltpu.VMEM((_BW, _DIM), jnp.float32),
            pltpu.VMEM((_NBH,), jnp.int32),
            pltpu.VMEM((_NBH,), jnp.int32),
            pltpu.VMEM((_NBW,), jnp.int32),
            pltpu.VMEM((_NBH, _DIM), jnp.float32),
            pltpu.SemaphoreType.DMA,
            pltpu.SemaphoreType.DMA,
            pltpu.SemaphoreType.DMA,
            pltpu.SemaphoreType.DMA,
        ],
    )
    def k(u_hbm, v_hbm, vg_hbm, vm_hbm, adje_hbm, adjr_hbm, usr_hbm, ent_hbm,
          uemb_out, self_out, nbrel_out, nbvec_out,
          uidx, vidx, vgidx, vmidx, adje_rows, adjr_rows, urows, srows,
          flat_a, flat_b, frel, rows,
          sem_u, sem_s, sem_a, sem_r):
        wid = lax.axis_index("s") * _NC + lax.axis_index("c")
        base = wid * _BW
        pltpu.sync_copy(u_hbm.at[pl.ds(base, _BW)], uidx)
        pltpu.sync_copy(v_hbm.at[pl.ds(base, _BW)], vidx)
        pltpu.sync_copy(vg_hbm.at[pl.ds(base, _BW)], vgidx)
        pltpu.sync_copy(vm_hbm.at[pl.ds(base, _BW)], vmidx)
        cu = pltpu.async_copy(usr_hbm.at[uidx], urows, sem_u)
        cs = pltpu.async_copy(ent_hbm.at[vidx], srows, sem_s)
        ca = pltpu.async_copy(adje_hbm.at[vgidx], adje_rows, sem_a)
        cb = pltpu.async_copy(adjr_hbm.at[vgidx], adjr_rows, sem_a)
        ca.wait()
        cb.wait()
        lane = lax.iota(jnp.int32, _L)
        for kk in range(_NNB):
            dst = flat_a if kk < _KH else flat_b
            for h in range(_BW // _L):
                rows_h = lane + h * _L
                off = plsc.load_gather(vmidx, [rows_h]) + kk
                vals = plsc.load_gather(adje_rows, [rows_h, off])
                dst[pl.ds((kk % _KH) * _BW + h * _L, _L)] = vals
        rowoff = lax.shift_right_logical(lane, 3)
        koff = lax.bitwise_and(lane, 7)
        for jj in range(_NBW // _L):
            rows_jj = rowoff + jj * 2
            off = plsc.load_gather(vmidx, [rows_jj]) + koff
            frel[pl.ds(jj * _L, _L)] = plsc.load_gather(adjr_rows, [rows_jj, off])
        cr = pltpu.async_copy(ent_hbm.at[flat_a], rows, sem_r)
        cu.wait()
        cs.wait()
        pltpu.sync_copy(urows, uemb_out.at[pl.ds(base, _BW)])
        pltpu.sync_copy(srows, self_out.at[pl.ds(base, _BW)])
        pltpu.sync_copy(frel, nbrel_out.at[pl.ds(wid * _NBW, _NBW)])
        cr.wait()
        for kk in range(_KH):
            pltpu.sync_copy(rows.at[pl.ds(kk * _BW, _BW)],
                            nbvec_out.at[kk, pl.ds(base, _BW)])
        pltpu.async_copy(ent_hbm.at[flat_b], rows, sem_r).wait()
        for kk in range(_KH):
            pltpu.sync_copy(rows.at[pl.ds(kk * _BW, _BW)],
                            nbvec_out.at[_KH + kk, pl.ds(base, _BW)])

    return k(u_pad, v_pad, vg_pad, vm_pad, adj_e8, adj_r8, usr, ent)


def _tc_compute(user_emb, self_vec, nb_vec, nb_rel, relT, W_aggT, W_linP):
    BM = 256
    grid = (_BP // BM,)

    def body(user_ref, self_ref, nb_ref, nbr_ref, relT_ref,
             wagg_ref, wlin_ref, fea_ref, feaa_ref):
        i = pl.program_id(0)
        user = user_ref[...]
        s_all = jnp.dot(user, relT_ref[...], preferred_element_type=jnp.float32)
        nbr = nbr_ref[...]
        r_iota = lax.broadcasted_iota(jnp.int32, (BM, _NRELP), 1)
        cols = []
        for kk in range(_NNB):
            sel = nbr[:, kk:kk + 1] == r_iota
            cols.append(jnp.sum(jnp.where(sel, s_all, 0.0), axis=1,
                                keepdims=True))
        scores = jnp.concatenate(cols, axis=1)
        m = jnp.max(scores, axis=-1, keepdims=True)
        e = jnp.exp(scores - m)
        w = e / jnp.sum(e, axis=-1, keepdims=True)
        agg = w[:, 0:1] * nb_ref[0]
        for kk in range(1, _NNB):
            agg = agg + w[:, kk:kk + 1] * nb_ref[kk]
        x = self_ref[...] + agg
        item = jnp.tanh(jnp.dot(x, wagg_ref[...],
                                preferred_element_type=jnp.float32))
        xp = x + jnp.sign(agg) * _EPS
        item2 = jnp.tanh(jnp.dot(xp, wagg_ref[...],
                                 preferred_element_type=jnp.float32))
        wl = wlin_ref[...]
        fa = jnp.dot(wl, item, preferred_element_type=jnp.float32)
        fb = jnp.dot(wl, item2, preferred_element_type=jnp.float32)

        @pl.when(i == 0)
        def _():
            fea_ref[...] = jnp.zeros_like(fea_ref)
            feaa_ref[...] = jnp.zeros_like(feaa_ref)

        fea_ref[...] += fa
        feaa_ref[...] += fb

    return pl.pallas_call(
        body,
        grid=grid,
        in_specs=[
            pl.BlockSpec((BM, _DIM), lambda i: (i, 0)),
            pl.BlockSpec((BM, _DIM), lambda i: (i, 0)),
            pl.BlockSpec((_NNB, BM, _DIM), lambda i: (0, i, 0)),
            pl.BlockSpec((BM, _NNB), lambda i: (i, 0)),
            pl.BlockSpec((_DIM, _NRELP), lambda i: (0, 0)),
            pl.BlockSpec((_DIM, _DIM), lambda i: (0, 0)),
            pl.BlockSpec((_NCTX, BM), lambda i: (0, i)),
        ],
        out_specs=[
            pl.BlockSpec((_NCTX, _DIM), lambda i: (0, 0)),
            pl.BlockSpec((_NCTX, _DIM), lambda i: (0, 0)),
        ],
        out_shape=[
            jax.ShapeDtypeStruct((_NCTX, _DIM), jnp.float32),
            jax.ShapeDtypeStruct((_NCTX, _DIM), jnp.float32),
        ],
    )(user_emb, self_vec, nb_vec, nb_rel, relT, W_aggT, W_linP)


def kernel(u, v, adj_ent, adj_rel, usr, ent, rel, W_agg, W_lin):
    bsz = u.shape[0]
    u_pad = jnp.zeros((_BP,), jnp.int32).at[:bsz].set(u.astype(jnp.int32))
    v_pad = jnp.zeros((_BP,), jnp.int32).at[:bsz].set(v.astype(jnp.int32))
    vg_pad = v_pad // _EPR
    vm_pad = (v_pad % _EPR) * _NNB
    adj_e8 = adj_ent.astype(jnp.int32).reshape(-1, 128)
    adj_r8 = adj_rel.astype(jnp.int32).reshape(-1, 128)

    uemb, selfv, nbrel_flat, nbvec = _sc_gather(
        u_pad, v_pad, vg_pad, vm_pad, adj_e8, adj_r8, usr, ent)

    nb_rel = nbrel_flat.reshape(_BP, _NNB)
    relT = jnp.zeros((_DIM, _NRELP), jnp.float32).at[:, :rel.shape[0]].set(rel.T)
    W_linP = jnp.zeros((_NCTX, _BP), jnp.float32).at[:, :bsz].set(W_lin)

    fea, fea_agg = _tc_compute(uemb, selfv, nbvec, nb_rel,
                               relT, W_agg.T, W_linP)
    return fea, fea_agg


# speedup vs baseline: 1.3675x; 1.3675x over previous
import functools

import jax
import jax.numpy as jnp
from jax import lax
from jax.experimental import pallas as pl
from jax.experimental.pallas import tpu as pltpu
from jax.experimental.pallas import tpu_sc as plsc

_NC = 2
_NS = 16
_NW = _NC * _NS
_L = 16

_B = 1000
_BP = 1024
_BW = _BP // _NW
_NNB = 8
_NBW = _BW * _NNB
_NBH = _NBW // 2
_KH = _NNB // 2
_DIM = 512
_EPS = 0.01
_NRELP = 64
_NCTX = 16
_EPR = 8


def _sc_gather(u_pad, v_pad, vg_pad, vm_pad, adj_cat, usr, ent):
    mesh = plsc.VectorSubcoreMesh(core_axis_name="c", subcore_axis_name="s")

    @functools.partial(
        pl.kernel,
        mesh=mesh,
        compiler_params=pltpu.CompilerParams(needs_layout_passes=False),
        out_type=(
            jax.ShapeDtypeStruct((_BP, _DIM), jnp.float32),
            jax.ShapeDtypeStruct((_BP, _DIM), jnp.float32),
            jax.ShapeDtypeStruct((_BP * _NNB,), jnp.int32),
            jax.ShapeDtypeStruct((_NNB, _BP, _DIM), jnp.float32),
        ),
        scratch_types=[
            pltpu.VMEM((_BW,), jnp.int32),
            pltpu.VMEM((_BW,), jnp.int32),
            pltpu.VMEM((_BW,), jnp.int32),
            pltpu.VMEM((_BW,), jnp.int32),
            pltpu.VMEM((_BW, 128), jnp.int32),
            pltpu.VMEM((_BW, _DIM), jnp.float32),
            pltpu.VMEM((_BW, _DIM), jnp.float32),
            pltpu.VMEM((_NBH,), jnp.int32),
            pltpu.VMEM((_NBH,), jnp.int32),
            pltpu.VMEM((_NBW,), jnp.int32),
            pltpu.VMEM((_NBH, _DIM), jnp.float32),
            pltpu.SemaphoreType.DMA,
            pltpu.SemaphoreType.DMA,
            pltpu.SemaphoreType.DMA,
            pltpu.SemaphoreType.DMA,
        ],
    )
    def k(u_hbm, v_hbm, vg_hbm, vm_hbm, adj_hbm, usr_hbm, ent_hbm,
          uemb_out, self_out, nbrel_out, nbvec_out,
          uidx, vidx, vgidx, vmidx, adj_rows, urows, srows,
          flat_a, flat_b, frel, rows,
          sem_u, sem_s, sem_a, sem_r):
        wid = lax.axis_index("s") * _NC + lax.axis_index("c")
        base = wid * _BW
        pltpu.sync_copy(u_hbm.at[pl.ds(base, _BW)], uidx)
        pltpu.sync_copy(v_hbm.at[pl.ds(base, _BW)], vidx)
        pltpu.sync_copy(vg_hbm.at[pl.ds(base, _BW)], vgidx)
        pltpu.sync_copy(vm_hbm.at[pl.ds(base, _BW)], vmidx)
        cu = pltpu.async_copy(usr_hbm.at[uidx], urows, sem_u)
        cs = pltpu.async_copy(ent_hbm.at[vidx], srows, sem_s)
        ca = pltpu.async_copy(adj_hbm.at[vgidx], adj_rows, sem_a)
        ca.wait()
        lane = lax.iota(jnp.int32, _L)
        for kk in range(_NNB):
            dst = flat_a if kk < _KH else flat_b
            for h in range(_BW // _L):
                rows_h = lane + h * _L
                off = plsc.load_gather(vmidx, [rows_h]) + kk
                vals = plsc.load_gather(adj_rows, [rows_h, off])
                dst[pl.ds((kk % _KH) * _BW + h * _L, _L)] = vals
        rowoff = lax.shift_right_logical(lane, 3)
        koff = lax.bitwise_and(lane, 7) + _NNB
        for jj in range(_NBW // _L):
            rows_jj = rowoff + jj * 2
            off = plsc.load_gather(vmidx, [rows_jj]) + koff
            frel[pl.ds(jj * _L, _L)] = plsc.load_gather(adj_rows, [rows_jj, off])
        cr = pltpu.async_copy(ent_hbm.at[flat_a], rows, sem_r)
        cu.wait()
        cs.wait()
        pltpu.sync_copy(urows, uemb_out.at[pl.ds(base, _BW)])
        pltpu.sync_copy(srows, self_out.at[pl.ds(base, _BW)])
        pltpu.sync_copy(frel, nbrel_out.at[pl.ds(wid * _NBW, _NBW)])
        cr.wait()
        for kk in range(_KH):
            pltpu.sync_copy(rows.at[pl.ds(kk * _BW, _BW)],
                            nbvec_out.at[kk, pl.ds(base, _BW)])
        pltpu.async_copy(ent_hbm.at[flat_b], rows, sem_r).wait()
        for kk in range(_KH):
            pltpu.sync_copy(rows.at[pl.ds(kk * _BW, _BW)],
                            nbvec_out.at[_KH + kk, pl.ds(base, _BW)])

    return k(u_pad, v_pad, vg_pad, vm_pad, adj_cat, usr, ent)


def _tc_compute(user_emb, self_vec, nb_vec, nb_rel, relT, W_aggT, W_linP):
    BM = 256
    grid = (_BP // BM,)

    def body(user_ref, self_ref, nb_ref, nbr_ref, relT_ref,
             wagg_ref, wlin_ref, fea_ref, feaa_ref):
        i = pl.program_id(0)
        user = user_ref[...]
        s_all = jnp.dot(user, relT_ref[...], preferred_element_type=jnp.float32)
        nbr = nbr_ref[...]
        r_iota = lax.broadcasted_iota(jnp.int32, (BM, _NRELP), 1)
        cols = []
        for kk in range(_NNB):
            sel = nbr[:, kk:kk + 1] == r_iota
            cols.append(jnp.sum(jnp.where(sel, s_all, 0.0), axis=1,
                                keepdims=True))
        scores = jnp.concatenate(cols, axis=1)
        m = jnp.max(scores, axis=-1, keepdims=True)
        e = jnp.exp(scores - m)
        w = e / jnp.sum(e, axis=-1, keepdims=True)
        agg = w[:, 0:1] * nb_ref[0]
        for kk in range(1, _NNB):
            agg = agg + w[:, kk:kk + 1] * nb_ref[kk]
        x = self_ref[...] + agg
        item = jnp.tanh(jnp.dot(x, wagg_ref[...],
                                preferred_element_type=jnp.float32))
        xp = x + jnp.sign(agg) * _EPS
        item2 = jnp.tanh(jnp.dot(xp, wagg_ref[...],
                                 preferred_element_type=jnp.float32))
        wl = wlin_ref[...]
        fa = jnp.dot(wl, item, preferred_element_type=jnp.float32)
        fb = jnp.dot(wl, item2, preferred_element_type=jnp.float32)

        @pl.when(i == 0)
        def _():
            fea_ref[...] = jnp.zeros_like(fea_ref)
            feaa_ref[...] = jnp.zeros_like(feaa_ref)

        fea_ref[...] += fa
        feaa_ref[...] += fb

    return pl.pallas_call(
        body,
        grid=grid,
        in_specs=[
            pl.BlockSpec((BM, _DIM), lambda i: (i, 0)),
            pl.BlockSpec((BM, _DIM), lambda i: (i, 0)),
            pl.BlockSpec((_NNB, BM, _DIM), lambda i: (0, i, 0)),
            pl.BlockSpec((BM, _NNB), lambda i: (i, 0)),
            pl.BlockSpec((_DIM, _NRELP), lambda i: (0, 0)),
            pl.BlockSpec((_DIM, _DIM), lambda i: (0, 0)),
            pl.BlockSpec((_NCTX, BM), lambda i: (0, i)),
        ],
        out_specs=[
            pl.BlockSpec((_NCTX, _DIM), lambda i: (0, 0)),
            pl.BlockSpec((_NCTX, _DIM), lambda i: (0, 0)),
        ],
        out_shape=[
            jax.ShapeDtypeStruct((_NCTX, _DIM), jnp.float32),
            jax.ShapeDtypeStruct((_NCTX, _DIM), jnp.float32),
        ],
    )(user_emb, self_vec, nb_vec, nb_rel, relT, W_aggT, W_linP)


def kernel(u, v, adj_ent, adj_rel, usr, ent, rel, W_agg, W_lin):
    bsz = u.shape[0]
    u_pad = jnp.zeros((_BP,), jnp.int32).at[:bsz].set(u.astype(jnp.int32))
    v_pad = jnp.zeros((_BP,), jnp.int32).at[:bsz].set(v.astype(jnp.int32))
    vg_pad = v_pad // _EPR
    vm_pad = (v_pad % _EPR) * 16
    adj_cat = jnp.concatenate(
        [adj_ent.astype(jnp.int32), adj_rel.astype(jnp.int32)],
        axis=1).reshape(-1, 128)

    uemb, selfv, nbrel_flat, nbvec = _sc_gather(
        u_pad, v_pad, vg_pad, vm_pad, adj_cat, usr, ent)

    nb_rel = nbrel_flat.reshape(_BP, _NNB)
    relT = jnp.zeros((_DIM, _NRELP), jnp.float32).at[:, :rel.shape[0]].set(rel.T)
    W_linP = jnp.zeros((_NCTX, _BP), jnp.float32).at[:, :bsz].set(W_lin)

    fea, fea_agg = _tc_compute(uemb, selfv, nbvec, nb_rel,
                               relT, W_agg.T, W_linP)
    return fea, fea_agg
